# baseline (speedup 1.0000x reference)
import functools

import jax
import jax.numpy as jnp
from jax import lax
from jax.experimental import pallas as pl
from jax.experimental.pallas import tpu as pltpu
from jax.experimental.pallas import tpu_sc as plsc

N = 100000
E = 1600000
IN_DIM = 128
OUT_DIM = 128
H = 32
R = 4

NC = 2
NS = 16
NW = NC * NS

N_HALF = N // NC
RPT = 3126
N_HALF_PAD = NS * RPT
DUMP = N_HALF

K = 1024
SUB = 8
CPT = 100
EPT = K * CPT
E_PAD = NS * EPT

BN = 2000
NB = N // BN
NBH = N_HALF // BN


def _leaky(v):
    return jnp.where(v >= 0, v, 0.01 * v)


def _tc1_body(x_ref, W0_ref, b0_ref, Wr_ref, br_ref, emb_ref, T_ref, U_ref):
    emb = jnp.dot(x_ref[...], W0_ref[...], preferred_element_type=jnp.float32)
    emb = emb + b0_ref[...]
    emb_ref[...] = emb
    for r in range(R):
        A = Wr_ref[r, :H, :]
        B = Wr_ref[r, H:, :]
        T_ref[r] = jnp.dot(emb, A, preferred_element_type=jnp.float32)
        U_ref[r] = jnp.dot(emb, B, preferred_element_type=jnp.float32) + br_ref[r]


def _tc1(x, W0, b0, Wr1, br1):
    return pl.pallas_call(
        _tc1_body,
        grid=(NB,),
        in_specs=[
            pl.BlockSpec((BN, IN_DIM), lambda i: (i, 0)),
            pl.BlockSpec((IN_DIM, H), lambda i: (0, 0)),
            pl.BlockSpec((1, H), lambda i: (0, 0)),
            pl.BlockSpec((R, 2 * H, H), lambda i: (0, 0, 0)),
            pl.BlockSpec((R, H), lambda i: (0, 0)),
        ],
        out_specs=(
            pl.BlockSpec((BN, H), lambda i: (i, 0)),
            pl.BlockSpec((R, BN, H), lambda i: (0, i, 0)),
            pl.BlockSpec((R, BN, H), lambda i: (0, i, 0)),
        ),
        out_shape=(
            jax.ShapeDtypeStruct((N, H), jnp.float32),
            jax.ShapeDtypeStruct((R, N, H), jnp.float32),
            jax.ShapeDtypeStruct((R, N, H), jnp.float32),
        ),
    )(x, W0, b0, Wr1, br1)


def _tc2_body(s_ref, Wr_ref, br_ref, T_ref, U_ref):
    emb = s_ref[0]
    for r in range(R):
        A = Wr_ref[r, :H, :]
        B = Wr_ref[r, H:, :]
        T_ref[r] = jnp.dot(emb, A, preferred_element_type=jnp.float32)
        U_ref[r] = jnp.dot(emb, B, preferred_element_type=jnp.float32) + br_ref[r]


def _tc2(s_pad, Wr2, br2):
    return pl.pallas_call(
        _tc2_body,
        grid=(NC, NBH),
        in_specs=[
            pl.BlockSpec((1, BN, H), lambda c, i: (c, i, 0)),
            pl.BlockSpec((R, 2 * H, H), lambda c, i: (0, 0, 0)),
            pl.BlockSpec((R, H), lambda c, i: (0, 0)),
        ],
        out_specs=(
            pl.BlockSpec((R, BN, H), lambda c, i: (0, c * NBH + i, 0)),
            pl.BlockSpec((R, BN, H), lambda c, i: (0, c * NBH + i, 0)),
        ),
        out_shape=(
            jax.ShapeDtypeStruct((R, N, H), jnp.float32),
            jax.ShapeDtypeStruct((R, N, H), jnp.float32),
        ),
    )(s_pad, Wr2, br2)


def _tc3_body(s_ref, emb0_ref, Wf1_ref, bf1_ref, Wf2_ref, bf2_ref, out_ref):
    z = jnp.concatenate([s_ref[0], emb0_ref[...]], axis=1)
    z = jnp.dot(z, Wf1_ref[...], preferred_element_type=jnp.float32) + bf1_ref[...]
    z = _leaky(z)
    o = jnp.dot(z, Wf2_ref[...], preferred_element_type=jnp.float32) + bf2_ref[...]
    out_ref[...] = _leaky(o)


def _tc3(s_pad, emb0, Wf1, bf1, Wf2, bf2):
    return pl.pallas_call(
        _tc3_body,
        grid=(NC, NBH),
        in_specs=[
            pl.BlockSpec((1, BN, H), lambda c, i: (c, i, 0)),
            pl.BlockSpec((BN, H), lambda c, i: (c * NBH + i, 0)),
            pl.BlockSpec((2 * H, H), lambda c, i: (0, 0)),
            pl.BlockSpec((1, H), lambda c, i: (0, 0)),
            pl.BlockSpec((H, OUT_DIM), lambda c, i: (0, 0)),
            pl.BlockSpec((1, OUT_DIM), lambda c, i: (0, 0)),
        ],
        out_specs=pl.BlockSpec((BN, OUT_DIM), lambda c, i: (c * NBH + i, 0)),
        out_shape=jax.ShapeDtypeStruct((N, OUT_DIM), jnp.float32),
    )(s_pad, emb0, Wf1, bf1, Wf2, bf2)


@functools.lru_cache(maxsize=None)
def _sc_layer_fn(interpret=False):
  mesh = plsc.VectorSubcoreMesh(core_axis_name="c", subcore_axis_name="s")

  @functools.partial(
    pl.kernel,
    out_type=jax.ShapeDtypeStruct((NC, N_HALF_PAD, H), jnp.float32),
    mesh=mesh,
    interpret=interpret,
    compiler_params=pltpu.CompilerParams(use_tc_tiling_on_sc=False,
                                         needs_layout_passes=False),
    scratch_types=[
        pltpu.VMEM((2, 3, K), jnp.int32),
        pltpu.VMEM((SUB + 1, 128), jnp.int32),
        pltpu.VMEM((SUB + 1, 128), jnp.int32),
        pltpu.VMEM((SUB + 1, 128), jnp.int32),
        pltpu.VMEM((2, 128, H), jnp.float32),
        pltpu.VMEM((2, 128, H), jnp.float32),
        pltpu.VMEM_SHARED((N_HALF_PAD, H), jnp.float32),
        pltpu.SemaphoreType.DMA,
        pltpu.SemaphoreType.DMA,
        pltpu.SemaphoreType.DMA,
        pltpu.SemaphoreType.DMA,
        pltpu.SemaphoreType.DMA,
    ],
  )
  def _sc_layer(Th, Uh, packh, zh, outh,
                ebuf, giT2, giU2, lcl, rT, rU, acc,
                gsem0, gsem1, ssem0, ssem1, esem):
    cid = lax.axis_index("c")
    sid = lax.axis_index("s")
    base = cid * N_HALF
    gsem = (gsem0, gsem1)
    ssem = (ssem0, ssem1)

    pltpu.sync_copy(zh.at[pl.ds(sid * RPT, RPT)], acc.at[pl.ds(sid * RPT, RPT)])
    plsc.subcore_barrier()

    pltpu.async_copy(packh.at[sid, 0], ebuf.at[0], esem)
    pltpu.async_copy(packh.at[sid, 1], ebuf.at[1], esem)

    lane = jax.lax.iota(jnp.int32, 16)

    def process(i, par):
        pltpu.make_async_copy(packh.at[sid, i], ebuf.at[par], esem).wait()

        def reg(j, cnt):
            off = j * 16
            r16 = ebuf[par, 0, pl.ds(off, 16)]
            t16 = ebuf[par, 1, pl.ds(off, 16)]
            c16 = ebuf[par, 2, pl.ds(off, 16)]
            lc = c16 - base
            ok = (lc >= 0) & (lc < N_HALF)
            oki = jnp.where(ok, 1, 0)
            inc = plsc.cumsum(oki)
            pos = jnp.where(ok, cnt + (inc - oki), K + 112 + lane)
            i0 = lax.shift_right_logical(pos, 7)
            i1 = pos & 127
            plsc.store_scatter(giT2, [i0, i1], t16 * N + r16)
            plsc.store_scatter(giU2, [i0, i1], t16 * N + c16)
            plsc.store_scatter(lcl, [i0, i1], lc)
            return cnt + jnp.sum(oki)

        cnt = lax.fori_loop(0, K // 16, reg, 0)

        @pl.when(i + 2 < CPT)
        def _():
            pltpu.async_copy(packh.at[sid, i + 2], ebuf.at[par], esem)

        def pad(g, c2):
            pos = cnt + g * 16 + lane
            i0 = lax.shift_right_logical(pos, 7)
            i1 = pos & 127
            plsc.store_scatter(giT2, [i0, i1], jnp.zeros((16,), jnp.int32))
            plsc.store_scatter(giU2, [i0, i1], jnp.zeros((16,), jnp.int32))
            plsc.store_scatter(lcl, [i0, i1], jnp.full((16,), DUMP, jnp.int32))
            return c2

        lax.fori_loop(0, 8, pad, 0)

        for s in range(SUB):
            p = s % 2
            q = (s - 1) % 2

            @pl.when(s * 128 < cnt)
            def _(s=s, p=p, q=q):
                if s >= 2:
                    pltpu.make_async_copy(rT.at[p], acc.at[lcl.at[s - 2]],
                                          ssem[p]).wait()
                    pltpu.make_async_copy(rU.at[p], acc.at[lcl.at[s - 2]],
                                          ssem[p]).wait()
                pltpu.async_copy(Th.at[giT2.at[s]], rT.at[p], gsem[p])
                pltpu.async_copy(Uh.at[giU2.at[s]], rU.at[p], gsem[p])
                if s >= 1:
                    pltpu.make_async_copy(Th.at[giT2.at[s - 1]],
                                          rT.at[q], gsem[q]).wait()
                    pltpu.make_async_copy(Uh.at[giU2.at[s - 1]],
                                          rU.at[q], gsem[q]).wait()
                    pltpu.async_copy(rT.at[q], acc.at[lcl.at[s - 1]],
                                     ssem[q], add=True)
                    pltpu.async_copy(rU.at[q], acc.at[lcl.at[s - 1]],
                                     ssem[q], add=True)

        for s in range(SUB):
            p = s % 2

            @pl.when((s * 128 < cnt) & ((s + 1) * 128 >= cnt))
            def _(s=s, p=p):
                pltpu.make_async_copy(Th.at[giT2.at[s]], rT.at[p],
                                      gsem[p]).wait()
                pltpu.make_async_copy(Uh.at[giU2.at[s]], rU.at[p],
                                      gsem[p]).wait()
                pltpu.async_copy(rT.at[p], acc.at[lcl.at[s]], ssem[p], add=True)
                pltpu.async_copy(rU.at[p], acc.at[lcl.at[s]], ssem[p], add=True)
                pltpu.make_async_copy(rT.at[p], acc.at[lcl.at[s]],
                                      ssem[p]).wait()
                pltpu.make_async_copy(rU.at[p], acc.at[lcl.at[s]],
                                      ssem[p]).wait()
                if s >= 1:
                    q2 = (s - 1) % 2
                    pltpu.make_async_copy(rT.at[q2], acc.at[lcl.at[s - 1]],
                                          ssem[q2]).wait()
                    pltpu.make_async_copy(rU.at[q2], acc.at[lcl.at[s - 1]],
                                          ssem[q2]).wait()

    def pair(j, carry):
        process(2 * j, 0)
        process(2 * j + 1, 1)
        return carry

    lax.fori_loop(0, CPT // 2, pair, 0)
    plsc.subcore_barrier()
    pltpu.sync_copy(acc.at[pl.ds(sid * RPT, RPT)],
                    outh.at[cid, pl.ds(sid * RPT, RPT)])

  return _sc_layer


def kernel(x, edge_index, edge_type, edge_time, pos,
           W0, b0, Wr1, br1, Wr2, br2, Wf1, bf1, Wf2, bf2):
    del edge_time, pos
    row = edge_index[0]
    col = edge_index[1]
    pad = E_PAD - E
    rowp = jnp.concatenate([row, jnp.zeros((pad,), jnp.int32)])
    typp = jnp.concatenate([edge_type, jnp.zeros((pad,), jnp.int32)])
    colp = jnp.concatenate([col, jnp.full((pad,), N, jnp.int32)])
    pack = (jnp.stack([rowp, typp, colp])
            .reshape(3, NS, CPT, K).transpose(1, 2, 0, 3))
    zeros_half = jnp.zeros((N_HALF_PAD, H), jnp.float32)

    emb0, T1, U1 = _tc1(x, W0, b0.reshape(1, H), Wr1, br1)
    sc = _sc_layer_fn()
    s1 = sc(T1.reshape(R * N, H), U1.reshape(R * N, H), pack, zeros_half)
    T2, U2 = _tc2(s1, Wr2, br2)
    s2 = sc(T2.reshape(R * N, H), U2.reshape(R * N, H), pack, zeros_half)
    return _tc3(s2, emb0, Wf1, bf1.reshape(1, H), Wf2, bf2.reshape(1, OUT_DIM))


# speedup vs baseline: 1.5312x; 1.5312x over previous
import functools

import jax
import jax.numpy as jnp
from jax import lax
from jax.experimental import pallas as pl
from jax.experimental.pallas import tpu as pltpu
from jax.experimental.pallas import tpu_sc as plsc

N = 100000
E = 1600000
IN_DIM = 128
OUT_DIM = 128
H = 32
R = 4

NC = 2
NS = 16
NW = NC * NS

N_HALF = N // NC
RPT = 3126
N_HALF_PAD = NS * RPT
DUMP = N_HALF

K = 2048
SUB = 16
CPT = 50
EPT = K * CPT
E_PAD = NS * EPT

BN = 2000
NB = N // BN
NBH = N_HALF // BN


def _leaky(v):
    return jnp.where(v >= 0, v, 0.01 * v)


def _tc1_body(x_ref, W0_ref, b0_ref, Wr_ref, br_ref, emb_ref, T_ref, U_ref):
    emb = jnp.dot(x_ref[...], W0_ref[...], preferred_element_type=jnp.float32)
    emb = emb + b0_ref[...]
    emb_ref[...] = emb
    for r in range(R):
        A = Wr_ref[r, :H, :]
        B = Wr_ref[r, H:, :]
        T_ref[r] = jnp.dot(emb, A, preferred_element_type=jnp.float32)
        U_ref[r] = jnp.dot(emb, B, preferred_element_type=jnp.float32) + br_ref[r]


def _tc1(x, W0, b0, Wr1, br1):
    return pl.pallas_call(
        _tc1_body,
        grid=(NB,),
        in_specs=[
            pl.BlockSpec((BN, IN_DIM), lambda i: (i, 0)),
            pl.BlockSpec((IN_DIM, H), lambda i: (0, 0)),
            pl.BlockSpec((1, H), lambda i: (0, 0)),
            pl.BlockSpec((R, 2 * H, H), lambda i: (0, 0, 0)),
            pl.BlockSpec((R, H), lambda i: (0, 0)),
        ],
        out_specs=(
            pl.BlockSpec((BN, H), lambda i: (i, 0)),
            pl.BlockSpec((R, BN, H), lambda i: (0, i, 0)),
            pl.BlockSpec((R, BN, H), lambda i: (0, i, 0)),
        ),
        out_shape=(
            jax.ShapeDtypeStruct((N, H), jnp.float32),
            jax.ShapeDtypeStruct((R, N, H), jnp.float32),
            jax.ShapeDtypeStruct((R, N, H), jnp.float32),
        ),
    )(x, W0, b0, Wr1, br1)


def _tc2_body(s_ref, Wr_ref, br_ref, T_ref, U_ref):
    emb = s_ref[0]
    for r in range(R):
        A = Wr_ref[r, :H, :]
        B = Wr_ref[r, H:, :]
        T_ref[r] = jnp.dot(emb, A, preferred_element_type=jnp.float32)
        U_ref[r] = jnp.dot(emb, B, preferred_element_type=jnp.float32) + br_ref[r]


def _tc2(s_pad, Wr2, br2):
    return pl.pallas_call(
        _tc2_body,
        grid=(NC, NBH),
        in_specs=[
            pl.BlockSpec((1, BN, H), lambda c, i: (c, i, 0)),
            pl.BlockSpec((R, 2 * H, H), lambda c, i: (0, 0, 0)),
            pl.BlockSpec((R, H), lambda c, i: (0, 0)),
        ],
        out_specs=(
            pl.BlockSpec((R, BN, H), lambda c, i: (0, c * NBH + i, 0)),
            pl.BlockSpec((R, BN, H), lambda c, i: (0, c * NBH + i, 0)),
        ),
        out_shape=(
            jax.ShapeDtypeStruct((R, N, H), jnp.float32),
            jax.ShapeDtypeStruct((R, N, H), jnp.float32),
        ),
    )(s_pad, Wr2, br2)


def _tc3_body(s_ref, emb0_ref, Wf1_ref, bf1_ref, Wf2_ref, bf2_ref, out_ref):
    z = jnp.concatenate([s_ref[0], emb0_ref[...]], axis=1)
    z = jnp.dot(z, Wf1_ref[...], preferred_element_type=jnp.float32) + bf1_ref[...]
    z = _leaky(z)
    o = jnp.dot(z, Wf2_ref[...], preferred_element_type=jnp.float32) + bf2_ref[...]
    out_ref[...] = _leaky(o)


def _tc3(s_pad, emb0, Wf1, bf1, Wf2, bf2):
    return pl.pallas_call(
        _tc3_body,
        grid=(NC, NBH),
        in_specs=[
            pl.BlockSpec((1, BN, H), lambda c, i: (c, i, 0)),
            pl.BlockSpec((BN, H), lambda c, i: (c * NBH + i, 0)),
            pl.BlockSpec((2 * H, H), lambda c, i: (0, 0)),
            pl.BlockSpec((1, H), lambda c, i: (0, 0)),
            pl.BlockSpec((H, OUT_DIM), lambda c, i: (0, 0)),
            pl.BlockSpec((1, OUT_DIM), lambda c, i: (0, 0)),
        ],
        out_specs=pl.BlockSpec((BN, OUT_DIM), lambda c, i: (c * NBH + i, 0)),
        out_shape=jax.ShapeDtypeStruct((N, OUT_DIM), jnp.float32),
    )(s_pad, emb0, Wf1, bf1, Wf2, bf2)


@functools.lru_cache(maxsize=None)
def _sc_layer_fn(interpret=False):
  mesh = plsc.VectorSubcoreMesh(core_axis_name="c", subcore_axis_name="s")

  @functools.partial(
    pl.kernel,
    out_type=jax.ShapeDtypeStruct((NC, N_HALF_PAD, H), jnp.float32),
    mesh=mesh,
    interpret=interpret,
    compiler_params=pltpu.CompilerParams(use_tc_tiling_on_sc=False,
                                         needs_layout_passes=False),
    scratch_types=[
        pltpu.VMEM((3, K), jnp.int32),
        pltpu.VMEM((SUB + 1, 128), jnp.int32),
        pltpu.VMEM((SUB + 1, 128), jnp.int32),
        pltpu.VMEM((SUB + 1, 128), jnp.int32),
        pltpu.VMEM((2, 128, H), jnp.float32),
        pltpu.VMEM((2, 128, H), jnp.float32),
        pltpu.VMEM_SHARED((N_HALF_PAD, H), jnp.float32),
        pltpu.SemaphoreType.DMA,
        pltpu.SemaphoreType.DMA,
        pltpu.SemaphoreType.DMA,
        pltpu.SemaphoreType.DMA,
        pltpu.SemaphoreType.DMA,
    ],
  )
  def _sc_layer(Th, Uh, packh, zh, outh,
                ebuf, giT2, giU2, lcl, rT, rU, acc,
                gsem0, gsem1, ssem0, ssem1, esem):
    cid = lax.axis_index("c")
    sid = lax.axis_index("s")
    base = cid * N_HALF
    gsem = (gsem0, gsem1)
    ssem = (ssem0, ssem1)

    pltpu.sync_copy(zh.at[pl.ds(sid * RPT, RPT)], acc.at[pl.ds(sid * RPT, RPT)])
    plsc.subcore_barrier()

    lane = jax.lax.iota(jnp.int32, 16)

    def process(i):
        pltpu.sync_copy(packh.at[sid, i], ebuf)

        def reg(j, cnt):
            off = j * 16
            r16 = ebuf[0, pl.ds(off, 16)]
            t16 = ebuf[1, pl.ds(off, 16)]
            c16 = ebuf[2, pl.ds(off, 16)]
            lc = c16 - base
            ok = (lc >= 0) & (lc < N_HALF)
            oki = jnp.where(ok, 1, 0)
            inc = plsc.cumsum(oki)
            pos = jnp.where(ok, cnt + (inc - oki), K + 112 + lane)
            i0 = lax.shift_right_logical(pos, 7)
            i1 = pos & 127
            plsc.store_scatter(giT2, [i0, i1], t16 * N + r16)
            plsc.store_scatter(giU2, [i0, i1], t16 * N + c16)
            plsc.store_scatter(lcl, [i0, i1], lc)
            return cnt + jnp.sum(oki)

        cnt = lax.fori_loop(0, K // 16, reg, 0)

        def pad(g, c2):
            pos = cnt + g * 16 + lane
            i0 = lax.shift_right_logical(pos, 7)
            i1 = pos & 127
            plsc.store_scatter(giT2, [i0, i1], jnp.zeros((16,), jnp.int32))
            plsc.store_scatter(giU2, [i0, i1], jnp.zeros((16,), jnp.int32))
            plsc.store_scatter(lcl, [i0, i1], jnp.full((16,), DUMP, jnp.int32))
            return c2

        lax.fori_loop(0, 8, pad, 0)

        for s in range(SUB):
            p = s % 2
            q = (s - 1) % 2

            @pl.when(s * 128 < cnt)
            def _(s=s, p=p, q=q):
                if s >= 2:
                    pltpu.make_async_copy(rT.at[p], acc.at[lcl.at[s - 2]],
                                          ssem[p]).wait()
                    pltpu.make_async_copy(rU.at[p], acc.at[lcl.at[s - 2]],
                                          ssem[p]).wait()
                pltpu.async_copy(Th.at[giT2.at[s]], rT.at[p], gsem[p])
                pltpu.async_copy(Uh.at[giU2.at[s]], rU.at[p], gsem[p])
                if s >= 1:
                    pltpu.make_async_copy(Th.at[giT2.at[s - 1]],
                                          rT.at[q], gsem[q]).wait()
                    pltpu.make_async_copy(Uh.at[giU2.at[s - 1]],
                                          rU.at[q], gsem[q]).wait()
                    pltpu.async_copy(rT.at[q], acc.at[lcl.at[s - 1]],
                                     ssem[q], add=True)
                    pltpu.async_copy(rU.at[q], acc.at[lcl.at[s - 1]],
                                     ssem[q], add=True)

        for s in range(SUB):
            p = s % 2

            @pl.when((s * 128 < cnt) & ((s + 1) * 128 >= cnt))
            def _(s=s, p=p):
                pltpu.make_async_copy(Th.at[giT2.at[s]], rT.at[p],
                                      gsem[p]).wait()
                pltpu.make_async_copy(Uh.at[giU2.at[s]], rU.at[p],
                                      gsem[p]).wait()
                pltpu.async_copy(rT.at[p], acc.at[lcl.at[s]], ssem[p], add=True)
                pltpu.async_copy(rU.at[p], acc.at[lcl.at[s]], ssem[p], add=True)
                pltpu.make_async_copy(rT.at[p], acc.at[lcl.at[s]],
                                      ssem[p]).wait()
                pltpu.make_async_copy(rU.at[p], acc.at[lcl.at[s]],
                                      ssem[p]).wait()
                if s >= 1:
                    q2 = (s - 1) % 2
                    pltpu.make_async_copy(rT.at[q2], acc.at[lcl.at[s - 1]],
                                          ssem[q2]).wait()
                    pltpu.make_async_copy(rU.at[q2], acc.at[lcl.at[s - 1]],
                                          ssem[q2]).wait()

    def pair(j, carry):
        process(j)
        return carry

    lax.fori_loop(0, CPT, pair, 0)
    plsc.subcore_barrier()
    pltpu.sync_copy(acc.at[pl.ds(sid * RPT, RPT)],
                    outh.at[cid, pl.ds(sid * RPT, RPT)])

  return _sc_layer


def kernel(x, edge_index, edge_type, edge_time, pos,
           W0, b0, Wr1, br1, Wr2, br2, Wf1, bf1, Wf2, bf2):
    del edge_time, pos
    row = edge_index[0]
    col = edge_index[1]
    pad = E_PAD - E
    rowp = jnp.concatenate([row, jnp.zeros((pad,), jnp.int32)])
    typp = jnp.concatenate([edge_type, jnp.zeros((pad,), jnp.int32)])
    colp = jnp.concatenate([col, jnp.full((pad,), N, jnp.int32)])
    pack = (jnp.stack([rowp, typp, colp])
            .reshape(3, NS, CPT, K).transpose(1, 2, 0, 3))
    zeros_half = jnp.zeros((N_HALF_PAD, H), jnp.float32)

    emb0, T1, U1 = _tc1(x, W0, b0.reshape(1, H), Wr1, br1)
    sc = _sc_layer_fn()
    s1 = sc(T1.reshape(R * N, H), U1.reshape(R * N, H), pack, zeros_half)
    T2, U2 = _tc2(s1, Wr2, br2)
    s2 = sc(T2.reshape(R * N, H), U2.reshape(R * N, H), pack, zeros_half)
    return _tc3(s2, emb0, Wf1, bf1.reshape(1, H), Wf2, bf2.reshape(1, OUT_DIM))
